# baseline (speedup 1.0000x reference)
import functools

import jax
import jax.numpy as jnp
from jax.experimental import pallas as pl
from jax.experimental.pallas import tpu as pltpu


_PACK = 8
_LANES = 128


def _mlp_body(x_ref, w1_ref, b1_ref, w2_ref, b2_ref, o_ref):
    x = x_ref[...]
    h = jnp.dot(x, w1_ref[...], preferred_element_type=jnp.float32)
    h = jnp.maximum(h + b1_ref[...], 0.0)
    y = jnp.dot(h, w2_ref[...], preferred_element_type=jnp.float32)
    o_ref[...] = jnp.maximum(y + b2_ref[...], 0.0)


@functools.partial(jax.jit, static_argnames=("block_rows",))
def _run(xp, w1b, b1b, w2b, b2b, block_rows):
    rows = xp.shape[0]
    grid = rows // block_rows
    return pl.pallas_call(
        _mlp_body,
        grid=(grid,),
        in_specs=[
            pl.BlockSpec((block_rows, _LANES), lambda i: (i, 0)),
            pl.BlockSpec((_LANES, _LANES), lambda i: (0, 0)),
            pl.BlockSpec((1, _LANES), lambda i: (0, 0)),
            pl.BlockSpec((_LANES, _LANES), lambda i: (0, 0)),
            pl.BlockSpec((1, _LANES), lambda i: (0, 0)),
        ],
        out_specs=pl.BlockSpec((block_rows, _LANES), lambda i: (i, 0)),
        out_shape=jax.ShapeDtypeStruct((rows, _LANES), jnp.float32),
        compiler_params=pltpu.CompilerParams(
            dimension_semantics=("arbitrary",),
        ),
    )(xp, w1b, b1b, w2b, b2b)


def kernel(edge_attr, W1, b1, W2, b2):
    e, d = edge_attr.shape
    eye = jnp.eye(_PACK, dtype=jnp.float32)
    w1b = jnp.kron(eye, W1.astype(jnp.float32))
    w2b = jnp.kron(eye, W2.astype(jnp.float32))
    b1b = jnp.tile(b1.astype(jnp.float32), _PACK).reshape(1, _LANES)
    b2b = jnp.tile(b2.astype(jnp.float32), _PACK).reshape(1, _LANES)
    rows = e // _PACK
    block_rows = next(br for br in (4000, 2000, 1000, 500, 200, 100, 8, 1)
                      if rows % br == 0)
    xp = edge_attr.reshape(rows, _LANES)
    out = _run(xp, w1b, b1b, w2b, b2b, block_rows=block_rows)
    return out.reshape(e, d)


# speedup vs baseline: 1.0422x; 1.0422x over previous
import functools

import jax
import jax.numpy as jnp
from jax.experimental import pallas as pl
from jax.experimental.pallas import tpu as pltpu


def _mlp_body(x_ref, w1_ref, b1_ref, w2_ref, b2_ref, o_ref):
    x = x_ref[...]
    h = jnp.dot(x, w1_ref[...], preferred_element_type=jnp.float32)
    h = jnp.maximum(h + b1_ref[...], 0.0)
    y = jnp.dot(h, w2_ref[...], preferred_element_type=jnp.float32)
    o_ref[...] = jnp.maximum(y + b2_ref[...], 0.0)


@functools.partial(jax.jit, static_argnames=("block_rows",))
def _run(x, w1, b1, w2, b2, block_rows):
    rows, d = x.shape
    grid = rows // block_rows
    return pl.pallas_call(
        _mlp_body,
        grid=(grid,),
        in_specs=[
            pl.BlockSpec((block_rows, d), lambda i: (i, 0)),
            pl.BlockSpec((d, d), lambda i: (0, 0)),
            pl.BlockSpec((1, d), lambda i: (0, 0)),
            pl.BlockSpec((d, d), lambda i: (0, 0)),
            pl.BlockSpec((1, d), lambda i: (0, 0)),
        ],
        out_specs=pl.BlockSpec((block_rows, d), lambda i: (i, 0)),
        out_shape=jax.ShapeDtypeStruct((rows, d), jnp.float32),
        compiler_params=pltpu.CompilerParams(
            dimension_semantics=("arbitrary",),
        ),
    )(x, w1, b1, w2, b2)


def kernel(edge_attr, W1, b1, W2, b2):
    e, d = edge_attr.shape
    block_rows = next(br for br in (25600, 12800, 6400, 1600, 800, 400, 8, 1)
                      if e % br == 0)
    out = _run(edge_attr, W1, b1.reshape(1, d), W2, b2.reshape(1, d),
               block_rows=block_rows)
    return out
